# Initial kernel scaffold; baseline (speedup 1.0000x reference)
import jax
import jax.numpy as jnp
from jax.experimental import pallas as pl


def kernel(inputs, token_table, pos_table):
    raise NotImplementedError("write your pallas kernel here")


# speedup vs baseline: 2.1071x; 2.1071x over previous
import functools

import jax
import jax.numpy as jnp
from jax import lax
from jax.experimental import pallas as pl
from jax.experimental.pallas import tpu as pltpu
from jax.experimental.pallas import tpu_sc as plsc


def kernel(inputs, token_table, pos_table):
    B, L = inputs.shape
    V, D = token_table.shape
    R = B * L

    info = plsc.get_sparse_core_info()
    NC, NS = info.num_cores, info.num_subcores
    NW = NC * NS

    rows_per_w = R // NW
    CH = 100
    n_chunks = rows_per_w // CH
    assert rows_per_w % L == 0 and L % CH == 0 and rows_per_w % CH == 0
    SPC = L // CH
    NV = D // 16

    idx = inputs.reshape(NW, n_chunks, CH).astype(jnp.int32)

    mesh = plsc.VectorSubcoreMesh(core_axis_name="c", subcore_axis_name="s")

    @functools.partial(
        pl.kernel,
        out_type=jax.ShapeDtypeStruct((NW * n_chunks, CH, D), jnp.float32),
        mesh=mesh,
        compiler_params=pltpu.CompilerParams(use_tc_tiling_on_sc=False),
        scratch_types=[
            pltpu.VMEM((n_chunks, CH), jnp.int32),
            pltpu.VMEM((L, D), jnp.float32),
            pltpu.VMEM((CH, D), jnp.float32),
        ],
    )
    def emb_kernel(idx_hbm, tab_hbm, pos_hbm, out_hbm, idx_v, pos_v, buf):
        wid = lax.axis_index("s") * NC + lax.axis_index("c")
        cbase = wid * n_chunks
        pltpu.sync_copy(idx_hbm.at[wid], idx_v)
        pltpu.sync_copy(pos_hbm, pos_v)

        @pl.loop(0, n_chunks)
        def _chunk(g):
            pltpu.sync_copy(tab_hbm.at[idx_v.at[g]], buf)
            pbase = lax.rem(g, SPC) * CH

            @pl.loop(0, CH)
            def _row(r):
                for c in range(NV):
                    sl = pl.ds(c * 16, 16)
                    buf[r, sl] = buf[r, sl] + pos_v[pbase + r, sl]

            pltpu.sync_copy(buf, out_hbm.at[cbase + g])

    out = emb_kernel(idx, token_table, pos_table)
    return out.reshape(B, L, D)


# speedup vs baseline: 4.2226x; 2.0040x over previous
import functools

import jax
import jax.numpy as jnp
from jax import lax
from jax.experimental import pallas as pl
from jax.experimental.pallas import tpu as pltpu
from jax.experimental.pallas import tpu_sc as plsc

_NBUF = 8
_LEAD = 4


def kernel(inputs, token_table, pos_table):
    B, L = inputs.shape
    V, D = token_table.shape
    R = B * L

    info = plsc.get_sparse_core_info()
    NC, NS = info.num_cores, info.num_subcores
    NW = NC * NS

    rows_per_w = R // NW
    CH = 100
    n_chunks = rows_per_w // CH
    assert rows_per_w % L == 0 and L % CH == 0 and rows_per_w % CH == 0
    assert n_chunks % _NBUF == 0
    SPC = L // CH
    NV = D // 16

    idx = inputs.reshape(NW, n_chunks, CH).astype(jnp.int32)

    mesh = plsc.VectorSubcoreMesh(core_axis_name="c", subcore_axis_name="s")

    @functools.partial(
        pl.kernel,
        out_type=jax.ShapeDtypeStruct((NW * n_chunks, CH, D), jnp.float32),
        mesh=mesh,
        compiler_params=pltpu.CompilerParams(use_tc_tiling_on_sc=False),
        scratch_types=[
            pltpu.VMEM((n_chunks, CH), jnp.int32),
            pltpu.VMEM((L, D), jnp.float32),
        ]
        + [pltpu.VMEM((CH, D), jnp.float32)] * _NBUF
        + [pltpu.SemaphoreType.DMA] * (2 * _NBUF),
    )
    def emb_kernel(idx_hbm, tab_hbm, pos_hbm, out_hbm, idx_v, pos_v, *rest):
        bufs = rest[:_NBUF]
        gsems = rest[_NBUF : 2 * _NBUF]
        osems = rest[2 * _NBUF : 3 * _NBUF]

        wid = lax.axis_index("s") * NC + lax.axis_index("c")
        cbase = wid * n_chunks
        pltpu.sync_copy(idx_hbm.at[wid], idx_v)
        pltpu.sync_copy(pos_hbm, pos_v)

        def start_gather(g, b):
            pltpu.async_copy(tab_hbm.at[idx_v.at[g]], bufs[b], gsems[b])

        def wait_gather(g, b):
            pltpu.make_async_copy(tab_hbm.at[idx_v.at[g]], bufs[b], gsems[b]).wait()

        def start_out(g, b):
            pltpu.async_copy(bufs[b], out_hbm.at[cbase + g], osems[b])

        def wait_out(g, b):
            pltpu.make_async_copy(bufs[b], out_hbm.at[cbase + g], osems[b]).wait()

        for b in range(_LEAD):
            start_gather(b, b)

        @pl.loop(0, n_chunks, step=_NBUF)
        def _outer(g0):
            for b in range(_NBUF):
                g = g0 + b
                bn = (b + _LEAD) % _NBUF
                if b < _LEAD:

                    @pl.when(g0 > 0)
                    def _():
                        wait_out(g - _LEAD, bn)

                    start_gather(g + _LEAD, bn)
                else:
                    wait_out(g - _LEAD, bn)

                    @pl.when(g0 < n_chunks - _NBUF)
                    def _():
                        start_gather(g + _LEAD, bn)

                wait_gather(g, b)
                pbase = lax.rem(g, SPC) * CH

                @pl.loop(0, CH, unroll=2)
                def _row(r):
                    for c in range(NV):
                        sl = pl.ds(c * 16, 16)
                        plsc.addupdate(bufs[b].at[r, sl], pos_v[pbase + r, sl])

                start_out(g, b)

        for i in range(_LEAD):
            g = n_chunks - _LEAD + i
            wait_out(g, g % _NBUF)

    out = emb_kernel(idx, token_table, pos_table)
    return out.reshape(B, L, D)


# speedup vs baseline: 4.2283x; 1.0013x over previous
import functools

import jax
import jax.numpy as jnp
from jax import lax
from jax.experimental import pallas as pl
from jax.experimental.pallas import tpu as pltpu
from jax.experimental.pallas import tpu_sc as plsc

_NBUF = 8
_LEAD = 4


def kernel(inputs, token_table, pos_table):
    B, L = inputs.shape
    V, D = token_table.shape
    R = B * L

    info = plsc.get_sparse_core_info()
    NC, NS = info.num_cores, info.num_subcores
    NW = NC * NS

    rows_per_w = R // NW
    CH = 100
    n_chunks = rows_per_w // CH
    assert rows_per_w % L == 0 and L % CH == 0 and rows_per_w % CH == 0
    assert n_chunks % _NBUF == 0
    SPC = L // CH
    NV = D // 16

    idx = inputs.reshape(NW, n_chunks, CH).astype(jnp.int32)

    mesh = plsc.VectorSubcoreMesh(core_axis_name="c", subcore_axis_name="s")

    @functools.partial(
        pl.kernel,
        out_type=jax.ShapeDtypeStruct((B, L, D), jnp.float32),
        mesh=mesh,
        compiler_params=pltpu.CompilerParams(use_tc_tiling_on_sc=False),
        scratch_types=[
            pltpu.VMEM((n_chunks, CH), jnp.int32),
            pltpu.VMEM((L, D), jnp.float32),
        ]
        + [pltpu.VMEM((CH, D), jnp.float32)] * _NBUF
        + [pltpu.SemaphoreType.DMA] * (2 * _NBUF),
    )
    def emb_kernel(idx_hbm, tab_hbm, pos_hbm, out_hbm, idx_v, pos_v, *rest):
        bufs = rest[:_NBUF]
        gsems = rest[_NBUF : 2 * _NBUF]
        osems = rest[2 * _NBUF : 3 * _NBUF]

        wid = lax.axis_index("s") * NC + lax.axis_index("c")
        cbase = wid * n_chunks
        pltpu.sync_copy(idx_hbm.at[wid], idx_v)
        pltpu.sync_copy(pos_hbm, pos_v)

        def start_gather(g, b):
            pltpu.async_copy(tab_hbm.at[idx_v.at[g]], bufs[b], gsems[b])

        def wait_gather(g, b):
            pltpu.make_async_copy(tab_hbm.at[idx_v.at[g]], bufs[b], gsems[b]).wait()

        def out_slice(g):
            gg = cbase + g
            return out_hbm.at[gg // SPC, pl.ds(lax.rem(gg, SPC) * CH, CH)]

        def start_out(g, b):
            pltpu.async_copy(bufs[b], out_slice(g), osems[b])

        def wait_out(g, b):
            pltpu.make_async_copy(bufs[b], out_slice(g), osems[b]).wait()

        for b in range(_LEAD):
            start_gather(b, b)

        @pl.loop(0, n_chunks, step=_NBUF)
        def _outer(g0):
            for b in range(_NBUF):
                g = g0 + b
                bn = (b + _LEAD) % _NBUF
                if b < _LEAD:

                    @pl.when(g0 > 0)
                    def _():
                        wait_out(g - _LEAD, bn)

                    start_gather(g + _LEAD, bn)
                else:
                    wait_out(g - _LEAD, bn)

                    @pl.when(g0 < n_chunks - _NBUF)
                    def _():
                        start_gather(g + _LEAD, bn)

                wait_gather(g, b)
                pbase = lax.rem(g, SPC) * CH

                @pl.loop(0, CH, unroll=2)
                def _row(r):
                    for c in range(NV):
                        sl = pl.ds(c * 16, 16)
                        plsc.addupdate(bufs[b].at[r, sl], pos_v[pbase + r, sl])

                start_out(g, b)

        for i in range(_LEAD):
            g = n_chunks - _LEAD + i
            wait_out(g, g % _NBUF)

    return emb_kernel(idx, token_table, pos_table)
